# Initial kernel scaffold; baseline (speedup 1.0000x reference)
import jax
import jax.numpy as jnp
from jax.experimental import pallas as pl


def kernel(x, edge_index, W1, b1, W2, b2, W3, b3, Wc, bc):
    raise NotImplementedError("write your pallas kernel here")


# speedup vs baseline: 26.7898x; 26.7898x over previous
import functools

import jax
import jax.numpy as jnp
from jax import lax
from jax.experimental import pallas as pl
from jax.experimental.pallas import tpu as pltpu
from jax.experimental.pallas import tpu_sc as plsc

_N_NODES = 10000
_N_PAD = 10240
_E = 320000
_NC, _NS = 2, 16
_CH1 = _E // _NS
_CH3 = _E // (_NC * _NS)
_SLICE = _N_PAD // _NS

_mesh = plsc.VectorSubcoreMesh(core_axis_name="c", subcore_axis_name="s")


@functools.partial(
    pl.kernel,
    mesh=_mesh,
    out_type=jax.ShapeDtypeStruct((_NC, _N_PAD), jnp.float32),
    scratch_types=[
        pltpu.VMEM((_CH1,), jnp.int32),
        pltpu.VMEM((_CH1,), jnp.float32),
        pltpu.VMEM((_CH3,), jnp.int32),
        pltpu.VMEM((_CH3,), jnp.int32),
        pltpu.VMEM((_CH3,), jnp.float32),
        pltpu.VMEM((_SLICE,), jnp.float32),
        pltpu.VMEM_SHARED((_N_PAD,), jnp.float32),
        pltpu.VMEM_SHARED((_N_PAD,), jnp.float32),
    ],
)
def _sc_edge_weights(src_hbm, dst_hbm, ones_hbm, zeros_hbm, out_hbm,
                     d1_v, one_v, d3_v, s3_v, r_v, sl_v, deg_sh, c_sh):
    cid = lax.axis_index("c")
    sid = lax.axis_index("s")

    @pl.when(sid == 0)
    def _zero():
        pltpu.sync_copy(zeros_hbm, deg_sh)
        pltpu.sync_copy(zeros_hbm, c_sh)

    pltpu.sync_copy(dst_hbm.at[pl.ds(sid * _CH1, _CH1)], d1_v)
    pltpu.sync_copy(ones_hbm, one_v)
    plsc.subcore_barrier()

    pltpu.sync_copy(one_v, deg_sh.at[d1_v], add=True)
    plsc.subcore_barrier()

    base = sid * _SLICE
    pltpu.sync_copy(deg_sh.at[pl.ds(base, _SLICE)], sl_v)
    for i in range(_SLICE // 16):
        v = sl_v[pl.ds(i * 16, 16)]
        sl_v[pl.ds(i * 16, 16)] = 1.0 / jnp.maximum(v, 1.0)
    pltpu.sync_copy(sl_v, deg_sh.at[pl.ds(base, _SLICE)])
    plsc.subcore_barrier()

    wid = cid * _NS + sid
    pltpu.sync_copy(dst_hbm.at[pl.ds(wid * _CH3, _CH3)], d3_v)
    pltpu.sync_copy(src_hbm.at[pl.ds(wid * _CH3, _CH3)], s3_v)
    pltpu.sync_copy(deg_sh.at[d3_v], r_v)
    pltpu.sync_copy(r_v, c_sh.at[s3_v], add=True)
    plsc.subcore_barrier()

    @pl.when(sid == 0)
    def _flush():
        pltpu.sync_copy(c_sh, out_hbm.at[cid])


def _tc_body(x_ref, w1_ref, b1_ref, w2_ref, b2_ref, w3_ref, b3_ref,
             wc_ref, bc_ref, c_ref, o_ref):
    dn = (((1,), (0,)), ((), ()))
    hp = lax.Precision.HIGHEST
    h = jnp.tanh(
        lax.dot_general(x_ref[...], w1_ref[...], dn,
                        preferred_element_type=jnp.float32, precision=hp)
        + b1_ref[...])
    h = jnp.tanh(
        lax.dot_general(h, w2_ref[...], dn,
                        preferred_element_type=jnp.float32, precision=hp)
        + b2_ref[...])
    h = jnp.tanh(
        lax.dot_general(h, w3_ref[...], dn,
                        preferred_element_type=jnp.float32, precision=hp)
        + b3_ref[...])
    c = c_ref[...]
    w = (c[0:1, :_N_NODES] + c[1:2, :_N_NODES]) * (1.0 / _N_NODES)
    agg = lax.dot_general(w, h, dn,
                          preferred_element_type=jnp.float32, precision=hp)
    z = lax.dot_general(agg, wc_ref[...], dn,
                        preferred_element_type=jnp.float32, precision=hp)
    z = z + bc_ref[...]
    o_ref[...] = 1.0 / (1.0 + jnp.exp(-z))


def kernel(x, edge_index, W1, b1, W2, b2, W3, b3, Wc, bc):
    src = edge_index[0]
    dst = edge_index[1]
    ones = jnp.ones((_CH1,), jnp.float32)
    zeros = jnp.zeros((_N_PAD,), jnp.float32)
    cpart = _sc_edge_weights(src, dst, ones, zeros)
    out = pl.pallas_call(
        _tc_body,
        out_shape=jax.ShapeDtypeStruct((1, 1), jnp.float32),
    )(x, W1, b1.reshape(1, -1), W2, b2.reshape(1, -1), W3,
      b3.reshape(1, -1), Wc, bc.reshape(1, 1), cpart)
    return out.reshape(1)


# speedup vs baseline: 32.8543x; 1.2264x over previous
import functools

import jax
import jax.numpy as jnp
from jax import lax
from jax.experimental import pallas as pl
from jax.experimental.pallas import tpu as pltpu
from jax.experimental.pallas import tpu_sc as plsc

_N_NODES = 10000
_N_PAD = 10240
_E = 320000
_NC, _NS = 2, 16
_CH1 = _E // _NS
_CH3 = _E // (_NC * _NS)
_SLICE = _N_PAD // _NS

_mesh = plsc.VectorSubcoreMesh(core_axis_name="c", subcore_axis_name="s")


@functools.partial(
    pl.kernel,
    mesh=_mesh,
    out_type=jax.ShapeDtypeStruct((_NC, _N_PAD), jnp.float32),
    scratch_types=[
        pltpu.VMEM((_CH1,), jnp.int32),
        pltpu.VMEM((_CH1,), jnp.float32),
        pltpu.VMEM((_CH3,), jnp.int32),
        pltpu.VMEM((_CH3,), jnp.int32),
        pltpu.VMEM((_CH3,), jnp.float32),
        pltpu.VMEM((_SLICE,), jnp.float32),
        pltpu.VMEM_SHARED((_N_PAD,), jnp.float32),
        pltpu.VMEM_SHARED((_N_PAD,), jnp.float32),
    ],
)
def _sc_edge_weights(src_hbm, dst_hbm, ones_hbm, zeros_hbm, out_hbm,
                     d1_v, one_v, d3_v, s3_v, r_v, sl_v, deg_sh, c_sh):
    cid = lax.axis_index("c")
    sid = lax.axis_index("s")

    @pl.when(sid == 0)
    def _zero():
        pltpu.sync_copy(zeros_hbm, deg_sh)
        pltpu.sync_copy(zeros_hbm, c_sh)

    pltpu.sync_copy(dst_hbm.at[pl.ds(sid * _CH1, _CH1)], d1_v)
    pltpu.sync_copy(ones_hbm, one_v)
    plsc.subcore_barrier()

    pltpu.sync_copy(one_v, deg_sh.at[d1_v], add=True)
    plsc.subcore_barrier()

    base = sid * _SLICE
    pltpu.sync_copy(deg_sh.at[pl.ds(base, _SLICE)], sl_v)
    for i in range(_SLICE // 16):
        v = sl_v[pl.ds(i * 16, 16)]
        sl_v[pl.ds(i * 16, 16)] = 1.0 / jnp.maximum(v, 1.0)
    pltpu.sync_copy(sl_v, deg_sh.at[pl.ds(base, _SLICE)])
    plsc.subcore_barrier()

    wid = cid * _NS + sid
    pltpu.sync_copy(dst_hbm.at[pl.ds(wid * _CH3, _CH3)], d3_v)
    pltpu.sync_copy(src_hbm.at[pl.ds(wid * _CH3, _CH3)], s3_v)
    pltpu.sync_copy(deg_sh.at[d3_v], r_v)
    pltpu.sync_copy(r_v, c_sh.at[s3_v], add=True)
    plsc.subcore_barrier()

    @pl.when(sid == 0)
    def _flush():
        pltpu.sync_copy(c_sh, out_hbm.at[cid])


def _tc_mlp_body(x_ref, w1_ref, b1_ref, w2_ref, b2_ref, w3_ref, b3_ref,
                 wc_ref, y_ref):
    dn = (((1,), (0,)), ((), ()))
    hp = lax.Precision.HIGHEST
    h = jnp.tanh(
        lax.dot_general(x_ref[...], w1_ref[...], dn,
                        preferred_element_type=jnp.float32) + b1_ref[...])
    h = jnp.tanh(
        lax.dot_general(h, w2_ref[...], dn,
                        preferred_element_type=jnp.float32, precision=hp)
        + b2_ref[...])
    h = jnp.tanh(
        lax.dot_general(h, w3_ref[...], dn,
                        preferred_element_type=jnp.float32, precision=hp)
        + b3_ref[...])
    y_ref[...] = lax.dot_general(h, wc_ref[...], dn,
                                 preferred_element_type=jnp.float32,
                                 precision=hp)


def _tc_final_body(c_ref, y_ref, bc_ref, o_ref):
    c = c_ref[...]
    w = (c[0:1, :_N_NODES] + c[1:2, :_N_NODES]) * (1.0 / _N_NODES)
    z = lax.dot_general(w, y_ref[...], (((1,), (0,)), ((), ())),
                        preferred_element_type=jnp.float32,
                        precision=lax.Precision.HIGHEST) + bc_ref[...]
    o_ref[...] = 1.0 / (1.0 + jnp.exp(-z))


def kernel(x, edge_index, W1, b1, W2, b2, W3, b3, Wc, bc):
    src = edge_index[0]
    dst = edge_index[1]
    ones = jnp.ones((_CH1,), jnp.float32)
    zeros = jnp.zeros((_N_PAD,), jnp.float32)
    cpart = _sc_edge_weights(src, dst, ones, zeros)
    y = pl.pallas_call(
        _tc_mlp_body,
        out_shape=jax.ShapeDtypeStruct((_N_NODES, 1), jnp.float32),
    )(x, W1, b1.reshape(1, -1), W2, b2.reshape(1, -1), W3,
      b3.reshape(1, -1), Wc)
    out = pl.pallas_call(
        _tc_final_body,
        out_shape=jax.ShapeDtypeStruct((1, 1), jnp.float32),
    )(cpart, y, bc.reshape(1, 1))
    return out.reshape(1)


# speedup vs baseline: 38.8073x; 1.1812x over previous
import functools

import jax
import jax.numpy as jnp
from jax import lax
from jax.experimental import pallas as pl
from jax.experimental.pallas import tpu as pltpu
from jax.experimental.pallas import tpu_sc as plsc

_N_NODES = 10000
_N_PAD = 10240
_E = 320000
_NC, _NS = 2, 16
_CH1 = _E // _NS
_CH3 = _E // (_NC * _NS)
_SLICE = _N_PAD // _NS

_mesh = plsc.VectorSubcoreMesh(core_axis_name="c", subcore_axis_name="s")


@functools.partial(
    pl.kernel,
    mesh=_mesh,
    out_type=jax.ShapeDtypeStruct((_NC, _N_PAD), jnp.float32),
    scratch_types=[
        pltpu.VMEM((_CH1,), jnp.int32),
        pltpu.VMEM((_CH1,), jnp.float32),
        pltpu.VMEM((_CH3,), jnp.int32),
        pltpu.VMEM((_CH3,), jnp.int32),
        pltpu.VMEM((_CH3,), jnp.float32),
        pltpu.VMEM((_SLICE,), jnp.float32),
        pltpu.VMEM_SHARED((_N_PAD,), jnp.float32),
        pltpu.VMEM_SHARED((_N_PAD,), jnp.float32),
    ],
)
def _sc_edge_weights(src_hbm, dst_hbm, ones_hbm, zeros_hbm, out_hbm,
                     d1_v, one_v, d3_v, s3_v, r_v, sl_v, deg_sh, c_sh):
    cid = lax.axis_index("c")
    sid = lax.axis_index("s")

    @pl.when(sid == 0)
    def _zero():
        pltpu.sync_copy(zeros_hbm, deg_sh)
        pltpu.sync_copy(zeros_hbm, c_sh)

    pltpu.sync_copy(dst_hbm.at[pl.ds(sid * _CH1, _CH1)], d1_v)
    pltpu.sync_copy(ones_hbm, one_v)
    plsc.subcore_barrier()

    pltpu.sync_copy(one_v, deg_sh.at[d1_v], add=True)
    plsc.subcore_barrier()

    base = sid * _SLICE
    pltpu.sync_copy(deg_sh.at[pl.ds(base, _SLICE)], sl_v)
    for i in range(_SLICE // 16):
        v = sl_v[pl.ds(i * 16, 16)]
        sl_v[pl.ds(i * 16, 16)] = 1.0 / jnp.maximum(v, 1.0)
    pltpu.sync_copy(sl_v, deg_sh.at[pl.ds(base, _SLICE)])
    plsc.subcore_barrier()

    wid = cid * _NS + sid
    pltpu.sync_copy(dst_hbm.at[pl.ds(wid * _CH3, _CH3)], d3_v)
    pltpu.sync_copy(src_hbm.at[pl.ds(wid * _CH3, _CH3)], s3_v)
    pltpu.sync_copy(deg_sh.at[d3_v], r_v)
    pltpu.sync_copy(r_v, c_sh.at[s3_v], add=True)
    plsc.subcore_barrier()

    @pl.when(sid == 0)
    def _flush():
        pltpu.sync_copy(c_sh, out_hbm.at[cid])


def _tc_mlp_body(x_ref, w1_ref, b1c_ref, w2t_ref, b2c_ref, w3t_ref, b3c_ref,
                 wct_ref, y_ref):
    dn = (((1,), (0,)), ((), ()))
    t1 = lax.dot_general(x_ref[...], w1_ref[...], dn,
                         preferred_element_type=jnp.float32)
    h = jnp.tanh(t1.T + b1c_ref[...])
    h = jnp.tanh(lax.dot_general(w2t_ref[...], h, dn,
                                 preferred_element_type=jnp.float32)
                 + b2c_ref[...])
    h = jnp.tanh(lax.dot_general(w3t_ref[...], h, dn,
                                 preferred_element_type=jnp.float32)
                 + b3c_ref[...])
    y_ref[...] = lax.dot_general(wct_ref[...], h, dn,
                                 preferred_element_type=jnp.float32)


def _tc_final_body(c_ref, y_ref, bc_ref, o_ref):
    c = c_ref[...]
    w = c[0:1, :_N_NODES] + c[1:2, :_N_NODES]
    z = (jnp.sum(w * y_ref[...], axis=1, keepdims=True) * (1.0 / _N_NODES)
         + bc_ref[...])
    o_ref[...] = 1.0 / (1.0 + jnp.exp(-z))


def kernel(x, edge_index, W1, b1, W2, b2, W3, b3, Wc, bc):
    src = edge_index[0]
    dst = edge_index[1]
    ones = jnp.ones((_CH1,), jnp.float32)
    zeros = jnp.zeros((_N_PAD,), jnp.float32)
    cpart = _sc_edge_weights(src, dst, ones, zeros)
    y = pl.pallas_call(
        _tc_mlp_body,
        out_shape=jax.ShapeDtypeStruct((1, _N_NODES), jnp.float32),
    )(x, W1, b1.reshape(-1, 1), W2.T, b2.reshape(-1, 1), W3.T,
      b3.reshape(-1, 1), Wc.T)
    out = pl.pallas_call(
        _tc_final_body,
        out_shape=jax.ShapeDtypeStruct((1, 1), jnp.float32),
    )(cpart, y, bc.reshape(1, 1))
    return out.reshape(1)


# speedup vs baseline: 50.5320x; 1.3021x over previous
import functools

import jax
import jax.numpy as jnp
from jax import lax
from jax.experimental import pallas as pl
from jax.experimental.pallas import tpu as pltpu
from jax.experimental.pallas import tpu_sc as plsc

_N_NODES = 10000
_N_PAD = 10240
_E = 320000
_NC, _NS = 2, 16
_CHE = _E // (_NC * _NS)
_SLICE = _N_PAD // _NS

_mesh = plsc.VectorSubcoreMesh(core_axis_name="c", subcore_axis_name="s")


@functools.partial(
    pl.kernel,
    mesh=_mesh,
    out_type=jax.ShapeDtypeStruct((_NC, _N_PAD), jnp.float32),
    scratch_types=[
        pltpu.VMEM((2, _CHE), jnp.int32),
        pltpu.VMEM((_CHE,), jnp.float32),
        pltpu.VMEM((_CHE,), jnp.float32),
        pltpu.VMEM((_SLICE,), jnp.float32),
        pltpu.VMEM_SHARED((_N_PAD,), jnp.float32),
        pltpu.VMEM_SHARED((_N_PAD,), jnp.float32),
    ],
    compiler_params=pltpu.CompilerParams(use_tc_tiling_on_sc=False),
)
def _sc_edge_weights(edge_hbm, out_hbm, ev, one_v, r_v, sl_v, deg_sh, c_sh):
    cid = lax.axis_index("c")
    sid = lax.axis_index("s")

    for i in range(_SLICE // 16):
        sl_v[pl.ds(i * 16, 16)] = jnp.zeros((16,), jnp.float32)
    pltpu.sync_copy(sl_v, deg_sh.at[pl.ds(sid * _SLICE, _SLICE)])
    pltpu.sync_copy(sl_v, c_sh.at[pl.ds(sid * _SLICE, _SLICE)])
    for i in range(_CHE // 16):
        one_v[pl.ds(i * 16, 16)] = jnp.full((16,), 1.0, jnp.float32)
    plsc.subcore_barrier()

    for r in range(2):
        base = (sid * 2 + r) * _CHE
        pltpu.sync_copy(edge_hbm.at[:, pl.ds(base, _CHE)], ev)
        pltpu.sync_copy(one_v, deg_sh.at[ev.at[1]], add=True)
    plsc.subcore_barrier()

    nbase = sid * _SLICE
    pltpu.sync_copy(deg_sh.at[pl.ds(nbase, _SLICE)], sl_v)
    for i in range(_SLICE // 16):
        v = sl_v[pl.ds(i * 16, 16)]
        sl_v[pl.ds(i * 16, 16)] = 1.0 / jnp.maximum(v, 1.0)
    pltpu.sync_copy(sl_v, deg_sh.at[pl.ds(nbase, _SLICE)])
    plsc.subcore_barrier()

    wid = cid * _NS + sid
    pltpu.sync_copy(edge_hbm.at[:, pl.ds(wid * _CHE, _CHE)], ev)
    pltpu.sync_copy(deg_sh.at[ev.at[1]], r_v)
    pltpu.sync_copy(r_v, c_sh.at[ev.at[0]], add=True)
    plsc.subcore_barrier()

    @pl.when(sid == 0)
    def _flush():
        pltpu.sync_copy(c_sh, out_hbm.at[cid])


def _tc_mlp_body(x_ref, w1_ref, b1c_ref, w2t_ref, b2c_ref, w3t_ref, b3c_ref,
                 wct_ref, y_ref):
    dn = (((1,), (0,)), ((), ()))
    t1 = lax.dot_general(x_ref[...], w1_ref[...], dn,
                         preferred_element_type=jnp.float32)
    h = jnp.tanh(t1.T + b1c_ref[...])
    h = jnp.tanh(lax.dot_general(w2t_ref[...], h, dn,
                                 preferred_element_type=jnp.float32)
                 + b2c_ref[...])
    h = jnp.tanh(lax.dot_general(w3t_ref[...], h, dn,
                                 preferred_element_type=jnp.float32)
                 + b3c_ref[...])
    y_ref[...] = lax.dot_general(wct_ref[...], h, dn,
                                 preferred_element_type=jnp.float32)


def _tc_final_body(c_ref, y_ref, bc_ref, o_ref):
    c = c_ref[...]
    w = c[0:1, :_N_NODES] + c[1:2, :_N_NODES]
    z = (jnp.sum(w * y_ref[...], axis=1, keepdims=True) * (1.0 / _N_NODES)
         + bc_ref[...])
    o_ref[...] = 1.0 / (1.0 + jnp.exp(-z))


def kernel(x, edge_index, W1, b1, W2, b2, W3, b3, Wc, bc):
    cpart = _sc_edge_weights(edge_index)
    y = pl.pallas_call(
        _tc_mlp_body,
        out_shape=jax.ShapeDtypeStruct((1, _N_NODES), jnp.float32),
    )(x, W1, b1.reshape(-1, 1), W2.T, b2.reshape(-1, 1), W3.T,
      b3.reshape(-1, 1), Wc.T)
    out = pl.pallas_call(
        _tc_final_body,
        out_shape=jax.ShapeDtypeStruct((1, 1), jnp.float32),
    )(cpart, y, bc.reshape(1, 1))
    return out.reshape(1)
